# baseline (speedup 1.0000x reference)
import functools

import jax
import jax.numpy as jnp
import numpy as np
from jax.experimental import pallas as pl
from jax.experimental.pallas import tpu as pltpu

B = 10000
S = 8
N_IN = 256
N_H = 64

B_BLK = 400
SUB = 16
N_SUB = B_BLK // SUB
GRID = B // B_BLK

_SEL = np.zeros((48, 128), dtype=np.float32)
for _i in range(16):
    _SEL[_i, _i * 8:_i * 8 + 7] = 1.0 / 7.0
    _SEL[16 + _i, _i * 8 + 7] = 1.0
    _SEL[32 + _i, _i * 8 + 6] = 1.0

_BDMASK = (np.arange(128)[None, :] // 8 == np.arange(16)[:, None]
           ).astype(np.float32)[None, :, None, :]


def _stage1_body(seq_ref, bd_ref, wcp_ref, sel_ref, wkbd_ref,
                 bias_ref, slope_ref, out1_ref, out2_ref, carry_ref):
    x = seq_ref[...].reshape(B_BLK * S, N_IN).astype(jnp.bfloat16)
    fts = jnp.dot(x, wcp_ref[...],
                  preferred_element_type=jnp.float32).astype(jnp.bfloat16)
    sel = sel_ref[...]
    wkbd = wkbd_ref[...]
    bias = bias_ref[...]
    slope = slope_ref[...]
    base = pl.program_id(0) * B_BLK
    for j in range(N_SUB):
        rows = slice(j * 128, (j + 1) * 128)
        h = jnp.dot(bd_ref[rows, :], fts[rows, :],
                    preferred_element_type=jnp.float32)
        y = h + bias
        hact = jnp.where(y >= 0, y, slope * y).astype(jnp.bfloat16)
        r = jnp.dot(sel, hact, preferred_element_type=jnp.float32)
        rb = r.astype(jnp.bfloat16)
        z12 = jnp.dot(rb[16:32, :], wkbd,
                      preferred_element_type=jnp.float32)
        o = slice(j * SUB, (j + 1) * SUB)
        out1_ref[o, 0:64] = rb[0:16, 0:64]
        out1_ref[o, 64:192] = z12.astype(jnp.bfloat16)
        out1_ref[o, 192:256] = rb[32:48, 64:128]
        ch = jnp.concatenate([rb[0:16, 0:64], rb[32:48, 64:128]], axis=1)
        ch_sh = jnp.concatenate([carry_ref[0:1, :], ch[0:15, :]], axis=0)
        out2_ref[pl.ds(base + j * SUB, SUB), :] = ch_sh
        carry_ref[0:1, :] = ch[15:16, :]
        if j == N_SUB - 1:
            @pl.when(pl.program_id(0) == GRID - 1)
            def _():
                out2_ref[0:1, :] = ch[14:15, :]


def _stage2_body(s1_ref, s2_ref, bk_ref, r1_ref, r2_ref):
    x = s1_ref[...]
    c = x[:, 0:64].astype(jnp.float32)
    z1 = x[:, 64:128].astype(jnp.float32)
    z2 = x[:, 128:192].astype(jnp.float32)
    han = x[:, 192:256].astype(jnp.float32)
    sh = s2_ref[...]
    csh = sh[:, 0:64].astype(jnp.float32)
    hsh = sh[:, 64:128].astype(jnp.float32)
    bkc = bk_ref[0, 0]
    bkp = bk_ref[0, 1]
    s0 = jnp.sum(z1 * c, axis=1, keepdims=True) + bkc
    s1 = jnp.sum(z1 * csh, axis=1, keepdims=True) + bkc
    r1_ref[...] = jnp.concatenate([s0, s1], axis=0)
    t0 = jnp.sum(z2 * han, axis=1, keepdims=True) + bkp
    t1 = jnp.sum(z2 * hsh, axis=1, keepdims=True) + bkp
    r2_ref[...] = jnp.concatenate([t0, t1], axis=0)


@functools.partial(jax.jit, static_argnames=("interpret",))
def _run(seq1, adj, Wc, bc, a_c, Wp, bp, a_p, Wk_c, bk_c, Wk_p, bk_p,
         interpret=False):
    adjr = adj.reshape(B // SUB, SUB, S, S)
    tiled = jnp.broadcast_to(adjr[:, :, :, None, :],
                             (B // SUB, SUB, S, SUB, S))
    tiled = tiled.reshape(B // SUB, SUB, S, 128)
    bd_rows = (tiled * jnp.asarray(_BDMASK)).astype(jnp.bfloat16)
    bd_rows = bd_rows.reshape(B * S, 128)

    wcp = jnp.concatenate([Wc.T, Wp.T], axis=1).astype(jnp.bfloat16)
    wkbd = jnp.zeros((128, 128), jnp.float32)
    wkbd = wkbd.at[0:64, 0:64].set(Wk_c).at[64:128, 64:128].set(Wk_p)
    wkbd = wkbd.astype(jnp.bfloat16)
    bias = jnp.concatenate([bc, bp])[None, :]
    slope = jnp.concatenate([jnp.broadcast_to(a_c, (64,)),
                             jnp.broadcast_to(a_p, (64,))])[None, :]
    bk = jnp.stack([bk_c[0], bk_p[0]])[None, :]
    sel = jnp.asarray(_SEL).astype(jnp.bfloat16)

    out1, out2 = pl.pallas_call(
        _stage1_body,
        grid=(GRID,),
        in_specs=[
            pl.BlockSpec((B_BLK, S, N_IN), lambda i: (i, 0, 0)),
            pl.BlockSpec((B_BLK * S, 128), lambda i: (i, 0)),
            pl.BlockSpec((N_IN, 128), lambda i: (0, 0)),
            pl.BlockSpec((48, 128), lambda i: (0, 0)),
            pl.BlockSpec((128, 128), lambda i: (0, 0)),
            pl.BlockSpec((1, 128), lambda i: (0, 0)),
            pl.BlockSpec((1, 128), lambda i: (0, 0)),
        ],
        out_specs=[
            pl.BlockSpec((B_BLK, 256), lambda i: (i, 0)),
            pl.BlockSpec((B, 128), lambda i: (0, 0)),
        ],
        out_shape=[
            jax.ShapeDtypeStruct((B, 256), jnp.bfloat16),
            jax.ShapeDtypeStruct((B, 128), jnp.bfloat16),
        ],
        scratch_shapes=[pltpu.VMEM((8, 128), jnp.bfloat16)],
        interpret=interpret,
    )(seq1, bd_rows, wcp, sel, wkbd, bias, slope)

    ret1, ret2 = pl.pallas_call(
        _stage2_body,
        out_shape=(jax.ShapeDtypeStruct((2 * B, 1), jnp.float32),
                   jax.ShapeDtypeStruct((2 * B, 1), jnp.float32)),
        interpret=interpret,
    )(out1, out2, bk)
    return ret1, ret2


def kernel(seq1, adj, Wc, bc, a_c, Wp, bp, a_p, Wk_c, bk_c, Wk_p, bk_p):
    return _run(seq1, adj, Wc, bc, a_c, Wp, bp, a_p,
                Wk_c, bk_c, Wk_p, bk_p)


# speedup vs baseline: 2.5073x; 2.5073x over previous
import functools

import jax
import jax.numpy as jnp
import numpy as np
from jax.experimental import pallas as pl
from jax.experimental.pallas import tpu as pltpu

B = 10000
S = 8
N_IN = 256
N_H = 64

B_BLK = 400
SUB = 16
N_SUB = B_BLK // SUB
GRID = B // B_BLK

_SEL = np.zeros((48, 128), dtype=np.float32)
for _i in range(16):
    _SEL[_i, _i * 8:_i * 8 + 7] = 1.0 / 7.0
    _SEL[16 + _i, _i * 8 + 7] = 1.0
    _SEL[32 + _i, _i * 8 + 6] = 1.0

_BDMASK = (np.arange(128)[None, :] // 8 == np.arange(16)[:, None]
           ).astype(np.float32)[None, :, None, :]


def _stage1_body(seq_ref, bd_ref, wcp_ref, sel_ref, wkbd_ref,
                 bias_ref, slope_ref, out1_ref, out2_ref, carry_ref):
    x = seq_ref[...].reshape(B_BLK * S, N_IN).astype(jnp.bfloat16)
    fts = jnp.dot(x, wcp_ref[...],
                  preferred_element_type=jnp.float32).astype(jnp.bfloat16)
    sel = sel_ref[...]
    wkbd = wkbd_ref[...]
    bias = bias_ref[...]
    slope = slope_ref[...]
    base = pl.program_id(0) * B_BLK
    for j in range(N_SUB):
        rows = slice(j * 128, (j + 1) * 128)
        h = jnp.dot(bd_ref[rows, :], fts[rows, :],
                    preferred_element_type=jnp.float32)
        y = h + bias
        hact = jnp.where(y >= 0, y, slope * y).astype(jnp.bfloat16)
        r = jnp.dot(sel, hact, preferred_element_type=jnp.float32)
        rb = r.astype(jnp.bfloat16)
        z12 = jnp.dot(rb[16:32, :], wkbd,
                      preferred_element_type=jnp.float32)
        o = slice(j * SUB, (j + 1) * SUB)
        out1_ref[o, 0:64] = rb[0:16, 0:64]
        out1_ref[o, 64:192] = z12.astype(jnp.bfloat16)
        out1_ref[o, 192:256] = rb[32:48, 64:128]
        ch = jnp.concatenate([rb[0:16, 0:64], rb[32:48, 64:128]], axis=1)
        ch_sh = jnp.concatenate([carry_ref[0:1, :], ch[0:15, :]], axis=0)
        out2_ref[pl.ds(base + j * SUB, SUB), :] = ch_sh
        carry_ref[0:1, :] = ch[15:16, :]
        if j == N_SUB - 1:
            @pl.when(pl.program_id(0) == GRID - 1)
            def _():
                out2_ref[0:1, :] = ch[14:15, :]


def _stage2_body(s1_ref, s2_ref, bk_ref, r1_ref, r2_ref):
    x = s1_ref[...]
    c = x[:, 0:64].astype(jnp.float32)
    z1 = x[:, 64:128].astype(jnp.float32)
    z2 = x[:, 128:192].astype(jnp.float32)
    han = x[:, 192:256].astype(jnp.float32)
    sh = s2_ref[...]
    csh = sh[:, 0:64].astype(jnp.float32)
    hsh = sh[:, 64:128].astype(jnp.float32)
    bkc = bk_ref[0, 0]
    bkp = bk_ref[0, 1]
    s0 = jnp.sum(z1 * c, axis=1, keepdims=True) + bkc
    s1 = jnp.sum(z1 * csh, axis=1, keepdims=True) + bkc
    r1_ref[...] = jnp.concatenate([s0, s1], axis=0)
    t0 = jnp.sum(z2 * han, axis=1, keepdims=True) + bkp
    t1 = jnp.sum(z2 * hsh, axis=1, keepdims=True) + bkp
    r2_ref[...] = jnp.concatenate([t0, t1], axis=0)


@functools.partial(jax.jit, static_argnames=("interpret",))
def _run(seq1, adj, Wc, bc, a_c, Wp, bp, a_p, Wk_c, bk_c, Wk_p, bk_p,
         interpret=False):
    adjb = adj.astype(jnp.bfloat16).reshape(B // SUB, SUB, S, S)
    bd_rows = jnp.concatenate(
        [jnp.pad(adjb[:, i], ((0, 0), (0, 0), (8 * i, 120 - 8 * i)))
         [:, None] for i in range(SUB)], axis=1)
    bd_rows = bd_rows.reshape(B * S, 128)

    wcp = jnp.concatenate([Wc.T, Wp.T], axis=1).astype(jnp.bfloat16)
    wkbd = jnp.zeros((128, 128), jnp.float32)
    wkbd = wkbd.at[0:64, 0:64].set(Wk_c).at[64:128, 64:128].set(Wk_p)
    wkbd = wkbd.astype(jnp.bfloat16)
    bias = jnp.concatenate([bc, bp])[None, :]
    slope = jnp.concatenate([jnp.broadcast_to(a_c, (64,)),
                             jnp.broadcast_to(a_p, (64,))])[None, :]
    bk = jnp.stack([bk_c[0], bk_p[0]])[None, :]
    sel = jnp.asarray(_SEL).astype(jnp.bfloat16)

    out1, out2 = pl.pallas_call(
        _stage1_body,
        grid=(GRID,),
        in_specs=[
            pl.BlockSpec((B_BLK, S, N_IN), lambda i: (i, 0, 0)),
            pl.BlockSpec((B_BLK * S, 128), lambda i: (i, 0)),
            pl.BlockSpec((N_IN, 128), lambda i: (0, 0)),
            pl.BlockSpec((48, 128), lambda i: (0, 0)),
            pl.BlockSpec((128, 128), lambda i: (0, 0)),
            pl.BlockSpec((1, 128), lambda i: (0, 0)),
            pl.BlockSpec((1, 128), lambda i: (0, 0)),
        ],
        out_specs=[
            pl.BlockSpec((B_BLK, 256), lambda i: (i, 0)),
            pl.BlockSpec((B, 128), lambda i: (0, 0)),
        ],
        out_shape=[
            jax.ShapeDtypeStruct((B, 256), jnp.bfloat16),
            jax.ShapeDtypeStruct((B, 128), jnp.bfloat16),
        ],
        scratch_shapes=[pltpu.VMEM((8, 128), jnp.bfloat16)],
        interpret=interpret,
    )(seq1, bd_rows, wcp, sel, wkbd, bias, slope)

    ret1, ret2 = pl.pallas_call(
        _stage2_body,
        out_shape=(jax.ShapeDtypeStruct((2 * B, 1), jnp.float32),
                   jax.ShapeDtypeStruct((2 * B, 1), jnp.float32)),
        interpret=interpret,
    )(out1, out2, bk)
    return ret1, ret2


def kernel(seq1, adj, Wc, bc, a_c, Wp, bp, a_p, Wk_c, bk_c, Wk_p, bk_p):
    return _run(seq1, adj, Wc, bc, a_c, Wp, bp, a_p,
                Wk_c, bk_c, Wk_p, bk_p)


# speedup vs baseline: 3.3812x; 1.3485x over previous
import functools

import jax
import jax.numpy as jnp
import numpy as np
from jax.experimental import pallas as pl
from jax.experimental.pallas import tpu as pltpu

B = 10000
S = 8
N_IN = 256
N_H = 64

B_BLK = 400
SUB = 16
N_SUB = B_BLK // SUB
GRID = B // B_BLK

_SEL = np.zeros((48, 128), dtype=np.float32)
for _i in range(16):
    _SEL[_i, _i * 8:_i * 8 + 7] = 1.0 / 7.0
    _SEL[16 + _i, _i * 8 + 7] = 1.0
    _SEL[32 + _i, _i * 8 + 6] = 1.0

_blk8 = np.arange(128) // 8
_MASK = (_blk8[:, None] == _blk8[None, :]).astype(np.float32)


def _stage1_body(seq_ref, bd_ref, mask_ref, wcp_ref, sel_ref, wkbd_ref,
                 bias_ref, slope_ref, out1_ref, out2_ref, carry_ref):
    x = seq_ref[...].reshape(B_BLK * S, N_IN).astype(jnp.bfloat16)
    fts = jnp.dot(x, wcp_ref[...],
                  preferred_element_type=jnp.float32).astype(jnp.bfloat16)
    sel = sel_ref[...]
    wkbd = wkbd_ref[...]
    bias = bias_ref[...]
    slope = slope_ref[...]
    base = pl.program_id(0) * B_BLK
    for j in range(N_SUB):
        rows = slice(j * 128, (j + 1) * 128)
        bd = bd_ref[rows, :] * mask_ref[...]
        h = jnp.dot(bd, fts[rows, :], preferred_element_type=jnp.float32)
        y = h + bias
        hact = jnp.where(y >= 0, y, slope * y).astype(jnp.bfloat16)
        r = jnp.dot(sel, hact, preferred_element_type=jnp.float32)
        rb = r.astype(jnp.bfloat16)
        z12 = jnp.dot(rb[16:32, :], wkbd,
                      preferred_element_type=jnp.float32)
        o = slice(j * SUB, (j + 1) * SUB)
        out1_ref[o, 0:64] = rb[0:16, 0:64]
        out1_ref[o, 64:192] = z12.astype(jnp.bfloat16)
        out1_ref[o, 192:256] = rb[32:48, 64:128]
        ch = jnp.concatenate([rb[0:16, 0:64], rb[32:48, 64:128]], axis=1)
        ch_sh = jnp.concatenate([carry_ref[0:1, :], ch[0:15, :]], axis=0)
        out2_ref[pl.ds(base + j * SUB, SUB), :] = ch_sh
        carry_ref[0:1, :] = ch[15:16, :]
        if j == N_SUB - 1:
            @pl.when(pl.program_id(0) == GRID - 1)
            def _():
                out2_ref[0:1, :] = ch[14:15, :]


def _stage2_body(s1_ref, s2_ref, bk_ref, r1_ref, r2_ref):
    x = s1_ref[...]
    c = x[:, 0:64].astype(jnp.float32)
    z1 = x[:, 64:128].astype(jnp.float32)
    z2 = x[:, 128:192].astype(jnp.float32)
    han = x[:, 192:256].astype(jnp.float32)
    sh = s2_ref[...]
    csh = sh[:, 0:64].astype(jnp.float32)
    hsh = sh[:, 64:128].astype(jnp.float32)
    bkc = bk_ref[0, 0]
    bkp = bk_ref[0, 1]
    s0 = jnp.sum(z1 * c, axis=1, keepdims=True) + bkc
    s1 = jnp.sum(z1 * csh, axis=1, keepdims=True) + bkc
    r1_ref[...] = jnp.concatenate([s0, s1], axis=0)
    t0 = jnp.sum(z2 * han, axis=1, keepdims=True) + bkp
    t1 = jnp.sum(z2 * hsh, axis=1, keepdims=True) + bkp
    r2_ref[...] = jnp.concatenate([t0, t1], axis=0)


@functools.partial(jax.jit, static_argnames=("interpret",))
def _run(seq1, adj, Wc, bc, a_c, Wp, bp, a_p, Wk_c, bk_c, Wk_p, bk_p,
         interpret=False):
    bd_rows = jnp.broadcast_to(adj.astype(jnp.bfloat16).reshape(B * S, 1, S),
                               (B * S, SUB, S)).reshape(B * S, 128)

    wcp = jnp.concatenate([Wc.T, Wp.T], axis=1).astype(jnp.bfloat16)
    wkbd = jnp.zeros((128, 128), jnp.float32)
    wkbd = wkbd.at[0:64, 0:64].set(Wk_c).at[64:128, 64:128].set(Wk_p)
    wkbd = wkbd.astype(jnp.bfloat16)
    bias = jnp.concatenate([bc, bp])[None, :]
    slope = jnp.concatenate([jnp.broadcast_to(a_c, (64,)),
                             jnp.broadcast_to(a_p, (64,))])[None, :]
    bk = jnp.stack([bk_c[0], bk_p[0]])[None, :]
    sel = jnp.asarray(_SEL).astype(jnp.bfloat16)
    mask = jnp.asarray(_MASK).astype(jnp.bfloat16)

    out1, out2 = pl.pallas_call(
        _stage1_body,
        grid=(GRID,),
        in_specs=[
            pl.BlockSpec((B_BLK, S, N_IN), lambda i: (i, 0, 0)),
            pl.BlockSpec((B_BLK * S, 128), lambda i: (i, 0)),
            pl.BlockSpec((128, 128), lambda i: (0, 0)),
            pl.BlockSpec((N_IN, 128), lambda i: (0, 0)),
            pl.BlockSpec((48, 128), lambda i: (0, 0)),
            pl.BlockSpec((128, 128), lambda i: (0, 0)),
            pl.BlockSpec((1, 128), lambda i: (0, 0)),
            pl.BlockSpec((1, 128), lambda i: (0, 0)),
        ],
        out_specs=[
            pl.BlockSpec((B_BLK, 256), lambda i: (i, 0)),
            pl.BlockSpec((B, 128), lambda i: (0, 0)),
        ],
        out_shape=[
            jax.ShapeDtypeStruct((B, 256), jnp.bfloat16),
            jax.ShapeDtypeStruct((B, 128), jnp.bfloat16),
        ],
        scratch_shapes=[pltpu.VMEM((8, 128), jnp.bfloat16)],
        interpret=interpret,
    )(seq1, bd_rows, mask, wcp, sel, wkbd, bias, slope)

    ret1, ret2 = pl.pallas_call(
        _stage2_body,
        out_shape=(jax.ShapeDtypeStruct((2 * B, 1), jnp.float32),
                   jax.ShapeDtypeStruct((2 * B, 1), jnp.float32)),
        interpret=interpret,
    )(out1, out2, bk)
    return ret1, ret2


def kernel(seq1, adj, Wc, bc, a_c, Wp, bp, a_p, Wk_c, bk_c, Wk_p, bk_p):
    return _run(seq1, adj, Wc, bc, a_c, Wp, bp, a_p,
                Wk_c, bk_c, Wk_p, bk_p)


# speedup vs baseline: 4.4303x; 1.3103x over previous
import functools

import jax
import jax.numpy as jnp
import numpy as np
from jax.experimental import pallas as pl
from jax.experimental.pallas import tpu as pltpu

B = 10000
S = 8
N_IN = 256
N_H = 64

B_BLK = 400
SUB = 16
N_SUB = B_BLK // SUB
GRID = B // B_BLK

_SEL = np.zeros((48, 128), dtype=np.float32)
for _i in range(16):
    _SEL[_i, _i * 8 + 1:_i * 8 + 8] = 1.0 / 7.0
    _SEL[16 + _i, _i * 8 + 0] = 1.0
    _SEL[32 + _i, _i * 8 + 1] = 1.0

_blk8 = np.arange(128) // 8
_MASK = (_blk8[:, None] == _blk8[None, :]).astype(np.float32)


def _stage1_body(seq_ref, adj_ref, wcp_ref, sel_ref, wkbd_ref,
                 bias_ref, slope_ref, mask8_ref, out1_ref, out2_ref,
                 carry_ref):
    x = seq_ref[...].reshape(B_BLK * S, N_IN).astype(jnp.bfloat16)
    fts = jnp.dot(x, wcp_ref[...],
                  preferred_element_type=jnp.float32).astype(jnp.bfloat16)
    sel = sel_ref[...]
    wkbd = wkbd_ref[...]
    bias = bias_ref[...]
    slope = slope_ref[...]
    mask8 = mask8_ref[...].reshape(1, 1, 128)
    base = pl.program_id(0) * B_BLK
    for j in range(N_SUB):
        rows = slice(j * 128, (j + 1) * 128)
        d16 = adj_ref[j * SUB:(j + 1) * SUB, :]
        u = jnp.broadcast_to(d16[:, None, :], (SUB, S, 64))
        v = jnp.pad(u, ((0, 0), (0, 0), (0, 64)))
        v = pltpu.roll(v, 8, 2, stride=8, stride_axis=1)
        v = v * mask8
        v = pltpu.roll(v, 64, 2)
        v = pltpu.roll(v, 0, 2, stride=8, stride_axis=0)
        bd = v.reshape(128, 128)
        h = jnp.dot(bd, fts[rows, :], preferred_element_type=jnp.float32)
        y = h + bias
        hact = jnp.where(y >= 0, y, slope * y).astype(jnp.bfloat16)
        r = jnp.dot(sel, hact, preferred_element_type=jnp.float32)
        rb = r.astype(jnp.bfloat16)
        z12 = jnp.dot(rb[16:32, :], wkbd,
                      preferred_element_type=jnp.float32)
        o = slice(j * SUB, (j + 1) * SUB)
        out1_ref[o, 0:64] = rb[0:16, 0:64]
        out1_ref[o, 64:192] = z12.astype(jnp.bfloat16)
        out1_ref[o, 192:256] = rb[32:48, 64:128]
        ch = jnp.concatenate([rb[0:16, 0:64], rb[32:48, 64:128]], axis=1)
        ch_sh = jnp.concatenate([carry_ref[0:1, :], ch[0:15, :]], axis=0)
        out2_ref[pl.ds(base + j * SUB, SUB), :] = ch_sh
        carry_ref[0:1, :] = ch[15:16, :]
        if j == N_SUB - 1:
            @pl.when(pl.program_id(0) == GRID - 1)
            def _():
                out2_ref[0:1, :] = ch[14:15, :]


def _stage2_body(s1_ref, s2_ref, bk_ref, r1_ref, r2_ref):
    x = s1_ref[...]
    c = x[:, 0:64].astype(jnp.float32)
    z1 = x[:, 64:128].astype(jnp.float32)
    z2 = x[:, 128:192].astype(jnp.float32)
    han = x[:, 192:256].astype(jnp.float32)
    sh = s2_ref[...]
    csh = sh[:, 0:64].astype(jnp.float32)
    hsh = sh[:, 64:128].astype(jnp.float32)
    bkc = bk_ref[0, 0]
    bkp = bk_ref[0, 1]
    s0 = jnp.sum(z1 * c, axis=1, keepdims=True) + bkc
    s1 = jnp.sum(z1 * csh, axis=1, keepdims=True) + bkc
    r1_ref[...] = jnp.concatenate([s0, s1], axis=0)
    t0 = jnp.sum(z2 * han, axis=1, keepdims=True) + bkp
    t1 = jnp.sum(z2 * hsh, axis=1, keepdims=True) + bkp
    r2_ref[...] = jnp.concatenate([t0, t1], axis=0)


@functools.partial(jax.jit, static_argnames=("interpret",))
def _run(seq1, adj, Wc, bc, a_c, Wp, bp, a_p, Wk_c, bk_c, Wk_p, bk_p,
         interpret=False):
    adjd = adj.astype(jnp.bfloat16).reshape(B, S * S)
    mask8 = jnp.asarray(((np.arange(128) >= 64) & (np.arange(128) < 72))
                        .astype(np.float32))[None, :].astype(jnp.bfloat16)

    wcp = jnp.concatenate([Wc.T, Wp.T], axis=1).astype(jnp.bfloat16)
    wkbd = jnp.zeros((128, 128), jnp.float32)
    wkbd = wkbd.at[0:64, 0:64].set(Wk_c).at[64:128, 64:128].set(Wk_p)
    wkbd = wkbd.astype(jnp.bfloat16)
    bias = jnp.concatenate([bc, bp])[None, :]
    slope = jnp.concatenate([jnp.broadcast_to(a_c, (64,)),
                             jnp.broadcast_to(a_p, (64,))])[None, :]
    bk = jnp.stack([bk_c[0], bk_p[0]])[None, :]
    sel = jnp.asarray(_SEL).astype(jnp.bfloat16)

    out1, out2 = pl.pallas_call(
        _stage1_body,
        grid=(GRID,),
        in_specs=[
            pl.BlockSpec((B_BLK, S, N_IN), lambda i: (i, 0, 0)),
            pl.BlockSpec((B_BLK, S * S), lambda i: (i, 0)),
            pl.BlockSpec((N_IN, 128), lambda i: (0, 0)),
            pl.BlockSpec((48, 128), lambda i: (0, 0)),
            pl.BlockSpec((128, 128), lambda i: (0, 0)),
            pl.BlockSpec((1, 128), lambda i: (0, 0)),
            pl.BlockSpec((1, 128), lambda i: (0, 0)),
            pl.BlockSpec((1, 128), lambda i: (0, 0)),
        ],
        out_specs=[
            pl.BlockSpec((B_BLK, 256), lambda i: (i, 0)),
            pl.BlockSpec((B, 128), lambda i: (0, 0)),
        ],
        out_shape=[
            jax.ShapeDtypeStruct((B, 256), jnp.bfloat16),
            jax.ShapeDtypeStruct((B, 128), jnp.bfloat16),
        ],
        scratch_shapes=[pltpu.VMEM((8, 128), jnp.bfloat16)],
        interpret=interpret,
    )(seq1, adjd, wcp, sel, wkbd, bias, slope, mask8)

    ret1, ret2 = pl.pallas_call(
        _stage2_body,
        out_shape=(jax.ShapeDtypeStruct((2 * B, 1), jnp.float32),
                   jax.ShapeDtypeStruct((2 * B, 1), jnp.float32)),
        interpret=interpret,
    )(out1, out2, bk)
    return ret1, ret2


def kernel(seq1, adj, Wc, bc, a_c, Wp, bp, a_p, Wk_c, bk_c, Wk_p, bk_p):
    return _run(seq1, adj, Wc, bc, a_c, Wp, bp, a_p,
                Wk_c, bk_c, Wk_p, bk_p)


# speedup vs baseline: 4.5219x; 1.0207x over previous
import functools

import jax
import jax.numpy as jnp
import numpy as np
from jax.experimental import pallas as pl
from jax.experimental.pallas import tpu as pltpu

B = 10000
S = 8
N_IN = 256
N_H = 64

B_BLK = 2000
SUB = 16
N_SUB = B_BLK // SUB
GRID = B // B_BLK

_SEL = np.zeros((48, 128), dtype=np.float32)
for _i in range(16):
    _SEL[_i, _i * 8 + 1:_i * 8 + 8] = 1.0 / 7.0
    _SEL[16 + _i, _i * 8 + 0] = 1.0
    _SEL[32 + _i, _i * 8 + 1] = 1.0

_blk8 = np.arange(128) // 8
_MASK = (_blk8[:, None] == _blk8[None, :]).astype(np.float32)


def _stage1_body(seq_ref, adj_ref, wcp_ref, sel_ref, wkbd_ref,
                 bias_ref, slope_ref, mask8_ref, out1_ref, out2_ref,
                 carry_ref):
    x = seq_ref[...].reshape(B_BLK * S, N_IN).astype(jnp.bfloat16)
    fts = jnp.dot(x, wcp_ref[...],
                  preferred_element_type=jnp.float32).astype(jnp.bfloat16)
    sel = sel_ref[...]
    wkbd = wkbd_ref[...]
    bias = bias_ref[...]
    slope = slope_ref[...]
    mask8 = mask8_ref[...].reshape(1, 1, 128)
    base = pl.program_id(0) * B_BLK
    for j in range(N_SUB):
        rows = slice(j * 128, (j + 1) * 128)
        d16 = adj_ref[j * SUB:(j + 1) * SUB, :]
        u = jnp.broadcast_to(d16[:, None, :], (SUB, S, 64))
        v = jnp.pad(u, ((0, 0), (0, 0), (0, 64)))
        v = pltpu.roll(v, 8, 2, stride=8, stride_axis=1)
        v = v * mask8
        v = pltpu.roll(v, 64, 2)
        v = pltpu.roll(v, 0, 2, stride=8, stride_axis=0)
        bd = v.reshape(128, 128)
        h = jnp.dot(bd, fts[rows, :], preferred_element_type=jnp.float32)
        y = h + bias
        hact = jnp.where(y >= 0, y, slope * y).astype(jnp.bfloat16)
        r = jnp.dot(sel, hact, preferred_element_type=jnp.float32)
        rb = r.astype(jnp.bfloat16)
        z12 = jnp.dot(rb[16:32, :], wkbd,
                      preferred_element_type=jnp.float32)
        o = slice(j * SUB, (j + 1) * SUB)
        out1_ref[o, 0:64] = rb[0:16, 0:64]
        out1_ref[o, 64:192] = z12.astype(jnp.bfloat16)
        out1_ref[o, 192:256] = rb[32:48, 64:128]
        ch = jnp.concatenate([rb[0:16, 0:64], rb[32:48, 64:128]], axis=1)
        ch_sh = jnp.concatenate([carry_ref[0:1, :], ch[0:15, :]], axis=0)
        out2_ref[pl.ds(base + j * SUB, SUB), :] = ch_sh
        carry_ref[0:1, :] = ch[15:16, :]
        if j == N_SUB - 1:
            @pl.when(pl.program_id(0) == GRID - 1)
            def _():
                out2_ref[0:1, :] = ch[14:15, :]


def _stage2_body(s1_ref, s2_ref, bk_ref, r1_ref, r2_ref):
    x = s1_ref[...]
    c = x[:, 0:64].astype(jnp.float32)
    z1 = x[:, 64:128].astype(jnp.float32)
    z2 = x[:, 128:192].astype(jnp.float32)
    han = x[:, 192:256].astype(jnp.float32)
    sh = s2_ref[...]
    csh = sh[:, 0:64].astype(jnp.float32)
    hsh = sh[:, 64:128].astype(jnp.float32)
    bkc = bk_ref[0, 0]
    bkp = bk_ref[0, 1]
    s0 = jnp.sum(z1 * c, axis=1, keepdims=True) + bkc
    s1 = jnp.sum(z1 * csh, axis=1, keepdims=True) + bkc
    r1_ref[...] = jnp.concatenate([s0, s1], axis=0)
    t0 = jnp.sum(z2 * han, axis=1, keepdims=True) + bkp
    t1 = jnp.sum(z2 * hsh, axis=1, keepdims=True) + bkp
    r2_ref[...] = jnp.concatenate([t0, t1], axis=0)


@functools.partial(jax.jit, static_argnames=("interpret",))
def _run(seq1, adj, Wc, bc, a_c, Wp, bp, a_p, Wk_c, bk_c, Wk_p, bk_p,
         interpret=False):
    adjd = adj.astype(jnp.bfloat16).reshape(B, S * S)
    mask8 = jnp.asarray(((np.arange(128) >= 64) & (np.arange(128) < 72))
                        .astype(np.float32))[None, :].astype(jnp.bfloat16)

    wcp = jnp.concatenate([Wc.T, Wp.T], axis=1).astype(jnp.bfloat16)
    wkbd = jnp.zeros((128, 128), jnp.float32)
    wkbd = wkbd.at[0:64, 0:64].set(Wk_c).at[64:128, 64:128].set(Wk_p)
    wkbd = wkbd.astype(jnp.bfloat16)
    bias = jnp.concatenate([bc, bp])[None, :]
    slope = jnp.concatenate([jnp.broadcast_to(a_c, (64,)),
                             jnp.broadcast_to(a_p, (64,))])[None, :]
    bk = jnp.stack([bk_c[0], bk_p[0]])[None, :]
    sel = jnp.asarray(_SEL).astype(jnp.bfloat16)

    out1, out2 = pl.pallas_call(
        _stage1_body,
        grid=(GRID,),
        in_specs=[
            pl.BlockSpec((B_BLK, S, N_IN), lambda i: (i, 0, 0)),
            pl.BlockSpec((B_BLK, S * S), lambda i: (i, 0)),
            pl.BlockSpec((N_IN, 128), lambda i: (0, 0)),
            pl.BlockSpec((48, 128), lambda i: (0, 0)),
            pl.BlockSpec((128, 128), lambda i: (0, 0)),
            pl.BlockSpec((1, 128), lambda i: (0, 0)),
            pl.BlockSpec((1, 128), lambda i: (0, 0)),
            pl.BlockSpec((1, 128), lambda i: (0, 0)),
        ],
        out_specs=[
            pl.BlockSpec((B_BLK, 256), lambda i: (i, 0)),
            pl.BlockSpec((B, 128), lambda i: (0, 0)),
        ],
        out_shape=[
            jax.ShapeDtypeStruct((B, 256), jnp.bfloat16),
            jax.ShapeDtypeStruct((B, 128), jnp.bfloat16),
        ],
        scratch_shapes=[pltpu.VMEM((8, 128), jnp.bfloat16)],
        interpret=interpret,
    )(seq1, adjd, wcp, sel, wkbd, bias, slope, mask8)

    ret1, ret2 = pl.pallas_call(
        _stage2_body,
        out_shape=(jax.ShapeDtypeStruct((2 * B, 1), jnp.float32),
                   jax.ShapeDtypeStruct((2 * B, 1), jnp.float32)),
        interpret=interpret,
    )(out1, out2, bk)
    return ret1, ret2


def kernel(seq1, adj, Wc, bc, a_c, Wp, bp, a_p, Wk_c, bk_c, Wk_p, bk_p):
    return _run(seq1, adj, Wc, bc, a_c, Wp, bp, a_p,
                Wk_c, bk_c, Wk_p, bk_p)


# speedup vs baseline: 6.2535x; 1.3829x over previous
import functools

import jax
import jax.numpy as jnp
import numpy as np
from jax.experimental import pallas as pl
from jax.experimental.pallas import tpu as pltpu

B = 10000
S = 8
N_IN = 256
N_H = 64

B_BLK = 2000
SUB = 16
N_SUB = B_BLK // SUB
GRID = B // B_BLK

_SEL = np.zeros((48, 128), dtype=np.float32)
for _i in range(16):
    _SEL[_i, _i * 8 + 1:_i * 8 + 8] = 1.0 / 7.0
    _SEL[16 + _i, _i * 8 + 0] = 1.0
    _SEL[32 + _i, _i * 8 + 1] = 1.0

_ONES2 = np.zeros((128, 2), dtype=np.float32)
_ONES2[0:64, 0] = 1.0
_ONES2[64:128, 1] = 1.0


def _body(seq_ref, adj_ref, wcp_ref, sel_ref, wkbd_ref, bias_ref,
          slope_ref, mask8_ref, ones2_ref, bk_ref, ret1_ref, ret2_ref,
          zscr, chscr, carry_ref, zrow0_ref):
    i = pl.program_id(0)
    base = i * B_BLK
    x = seq_ref[...].reshape(B_BLK * S, N_IN).astype(jnp.bfloat16)
    fts = jnp.dot(x, wcp_ref[...],
                  preferred_element_type=jnp.float32).astype(jnp.bfloat16)
    sel = sel_ref[...]
    bias = bias_ref[...]
    slope = slope_ref[...]
    mask8 = mask8_ref[...].reshape(1, 1, 128)
    for j in range(N_SUB):
        rows = slice(j * 128, (j + 1) * 128)
        d16 = adj_ref[j * SUB:(j + 1) * SUB, :]
        u = jnp.broadcast_to(d16[:, None, :], (SUB, S, 64))
        v = jnp.pad(u, ((0, 0), (0, 0), (0, 64)))
        v = pltpu.roll(v, 8, 2, stride=8, stride_axis=1)
        v = v * mask8
        v = pltpu.roll(v, 64, 2)
        v = pltpu.roll(v, 0, 2, stride=8, stride_axis=0)
        bd = v.reshape(128, 128)
        h = jnp.dot(bd, fts[rows, :], preferred_element_type=jnp.float32)
        y = h.astype(jnp.bfloat16) + bias
        hact = jnp.where(y >= 0, y, slope * y)
        r = jnp.dot(sel, hact, preferred_element_type=jnp.float32)
        rb = r.astype(jnp.bfloat16)
        o = slice(j * SUB, (j + 1) * SUB)
        zscr[o, :] = rb[16:32, :]
        chscr[o, :] = jnp.concatenate([rb[0:16, 0:64], rb[32:48, 64:128]],
                                      axis=1)

    zb = jnp.dot(zscr[...], wkbd_ref[...],
                 preferred_element_type=jnp.float32).astype(jnp.bfloat16)
    ch = chscr[...]
    ones2 = ones2_ref[...]
    bkc = bk_ref[0, 0]
    bkp = bk_ref[0, 1]

    @pl.when(i == 0)
    def _():
        zrow0_ref[0:1, :] = zb[0:1, :]

    st0 = jnp.dot(zb * ch, ones2, preferred_element_type=jnp.float32)
    chsh = jnp.concatenate([carry_ref[0:1, :], ch[0:B_BLK - 1, :]], axis=0)
    st1 = jnp.dot(zb * chsh, ones2, preferred_element_type=jnp.float32)
    carry_ref[0:1, :] = ch[B_BLK - 1:B_BLK, :]
    ret1_ref[pl.ds(base, B_BLK), :] = st0[:, 0:1] + bkc
    ret2_ref[pl.ds(base, B_BLK), :] = st0[:, 1:2] + bkp
    ret1_ref[pl.ds(B + base, B_BLK), :] = st1[:, 0:1] + bkc
    ret2_ref[pl.ds(B + base, B_BLK), :] = st1[:, 1:2] + bkp

    @pl.when(i == GRID - 1)
    def _():
        pz = zrow0_ref[0:1, :] * chscr[B_BLK - 2:B_BLK - 1, :]
        sw = jnp.dot(pz, ones2, preferred_element_type=jnp.float32)
        ret1_ref[B:B + 1, :] = sw[:, 0:1] + bkc
        ret2_ref[B:B + 1, :] = sw[:, 1:2] + bkp


@functools.partial(jax.jit, static_argnames=("interpret",))
def _run(seq1, adj, Wc, bc, a_c, Wp, bp, a_p, Wk_c, bk_c, Wk_p, bk_p,
         interpret=False):
    adjd = adj.astype(jnp.bfloat16).reshape(B, S * S)
    mask8 = jnp.asarray(((np.arange(128) >= 64) & (np.arange(128) < 72))
                        .astype(np.float32))[None, :].astype(jnp.bfloat16)

    wcp = jnp.concatenate([Wc.T, Wp.T], axis=1).astype(jnp.bfloat16)
    wkbd = jnp.zeros((128, 128), jnp.float32)
    wkbd = wkbd.at[0:64, 0:64].set(Wk_c).at[64:128, 64:128].set(Wk_p)
    wkbd = wkbd.astype(jnp.bfloat16)
    bias = jnp.concatenate([bc, bp])[None, :].astype(jnp.bfloat16)
    slope = jnp.concatenate([jnp.broadcast_to(a_c, (64,)),
                             jnp.broadcast_to(a_p, (64,))]
                            )[None, :].astype(jnp.bfloat16)
    bk = jnp.stack([bk_c[0], bk_p[0]])[None, :]
    sel = jnp.asarray(_SEL).astype(jnp.bfloat16)
    ones2 = jnp.asarray(_ONES2).astype(jnp.bfloat16)

    ret1, ret2 = pl.pallas_call(
        _body,
        grid=(GRID,),
        in_specs=[
            pl.BlockSpec((B_BLK, S, N_IN), lambda i: (i, 0, 0)),
            pl.BlockSpec((B_BLK, S * S), lambda i: (i, 0)),
            pl.BlockSpec((N_IN, 128), lambda i: (0, 0)),
            pl.BlockSpec((48, 128), lambda i: (0, 0)),
            pl.BlockSpec((128, 128), lambda i: (0, 0)),
            pl.BlockSpec((1, 128), lambda i: (0, 0)),
            pl.BlockSpec((1, 128), lambda i: (0, 0)),
            pl.BlockSpec((1, 128), lambda i: (0, 0)),
            pl.BlockSpec((128, 2), lambda i: (0, 0)),
            pl.BlockSpec((1, 2), lambda i: (0, 0)),
        ],
        out_specs=[
            pl.BlockSpec((2 * B, 1), lambda i: (0, 0)),
            pl.BlockSpec((2 * B, 1), lambda i: (0, 0)),
        ],
        out_shape=[
            jax.ShapeDtypeStruct((2 * B, 1), jnp.float32),
            jax.ShapeDtypeStruct((2 * B, 1), jnp.float32),
        ],
        scratch_shapes=[
            pltpu.VMEM((B_BLK, 128), jnp.bfloat16),
            pltpu.VMEM((B_BLK, 128), jnp.bfloat16),
            pltpu.VMEM((8, 128), jnp.bfloat16),
            pltpu.VMEM((8, 128), jnp.bfloat16),
        ],
        interpret=interpret,
    )(seq1, adjd, wcp, sel, wkbd, bias, slope, mask8, ones2, bk)
    return ret1, ret2


def kernel(seq1, adj, Wc, bc, a_c, Wp, bp, a_p, Wk_c, bk_c, Wk_p, bk_p):
    return _run(seq1, adj, Wc, bc, a_c, Wp, bp, a_p,
                Wk_c, bk_c, Wk_p, bk_p)
